# baseline (speedup 1.0000x reference)
import functools
import jax
import jax.numpy as jnp
from jax import lax
from jax.experimental import pallas as pl
from jax.experimental.pallas import tpu as pltpu
from jax.experimental.pallas import tpu_sc as plsc

_N = 10000
_E = 160000
_NS = 32
_LANE = 128
_NP = 10240
_P = 40
_NC = 2
_NSC = 16


def _zero_zbuf(zbuf):
    for rr in range(zbuf.shape[0]):
        for cc in range(zbuf.shape[1] // 16):
            zbuf[rr, pl.ds(cc * 16, 16)] = jnp.zeros((16,), jnp.float32)


def _sc_gather32(x32, src3g, dst3g):
    rows_per_tile = _E // (_NC * _NSC) // _P

    def body(x32_ref, s3_ref, d3_ref, xs_ref, xd_ref, sidx, didx, rbuf, sem):
        core = lax.axis_index("c")
        sid = lax.axis_index("s")
        wid = sid * _NC + core
        row0 = wid * rows_per_tile
        pltpu.sync_copy(s3_ref.at[wid], sidx)
        pltpu.sync_copy(d3_ref.at[wid], didx)

        def piece(k, _):
            pltpu.async_copy(x32_ref.at[sidx.at[k]], rbuf, sem).wait()
            pltpu.sync_copy(rbuf, xs_ref.at[row0 + k])
            pltpu.async_copy(x32_ref.at[didx.at[k]], rbuf, sem).wait()
            pltpu.sync_copy(rbuf, xd_ref.at[row0 + k])
            return 0

        lax.fori_loop(0, rows_per_tile, piece, 0)

    mesh = plsc.VectorSubcoreMesh(core_axis_name="c", subcore_axis_name="s")
    return pl.kernel(
        body,
        out_type=[jax.ShapeDtypeStruct((_E // _P, _P, _LANE), jnp.float32)] * 2,
        mesh=mesh,
        scratch_types=[
            pltpu.VMEM((rows_per_tile, _P), jnp.int32),
            pltpu.VMEM((rows_per_tile, _P), jnp.int32),
            pltpu.VMEM((_P, _LANE), jnp.float32),
            pltpu.SemaphoreType.DMA,
        ],
    )(x32, src3g, dst3g)


def _sc_scatter(h_chunks, ws_chunks, src3s, dst3s):
    nchunk = len(h_chunks)
    pieces = _E // _NSC // _P
    nsec = 5
    sec_p = pieces // nsec
    rows_dump = _NP // _NSC

    def body(*refs):
        h_refs = refs[:nchunk]
        ws_refs = refs[nchunk:2 * nchunk]
        s4_ref, d4_ref = refs[2 * nchunk:2 * nchunk + 2]
        out_refs = refs[2 * nchunk + 2:3 * nchunk + 2]
        (sidx, didx, hbuf, wbuf, zbuf, acc,
         gsem, wsem, asem) = refs[3 * nchunk + 2:]
        core = lax.axis_index("c")
        sid = lax.axis_index("s")
        _zero_zbuf(zbuf)

        for c in range(nchunk):
            @pl.when(core == (c % _NC))
            def _(c=c):
                def zrow(k, _):
                    pltpu.sync_copy(
                        zbuf, acc.at[pl.ds(sid * rows_dump + k * 16, 16), :])
                    return 0
                lax.fori_loop(0, rows_dump // 16, zrow, 0)
                plsc.subcore_barrier()

                for sec in range(nsec):
                    pltpu.sync_copy(s4_ref.at[sid, sec], sidx)
                    pltpu.sync_copy(d4_ref.at[sid, sec], didx)
                    pltpu.async_copy(
                        h_refs[c].at[sidx.at[0]], hbuf.at[0], gsem.at[0])
                    pltpu.async_copy(
                        ws_refs[c].at[sid * pieces + sec * sec_p],
                        wbuf.at[0], wsem.at[0])

                    def piece(k, _, sec=sec):
                        s = lax.rem(k, 2)
                        o = 1 - s
                        gp = sid * pieces + sec * sec_p + k

                        @pl.when(k >= 1)
                        def _():
                            pltpu.make_async_copy(
                                hbuf.at[o], acc.at[didx.at[k - 1]],
                                asem.at[o]).wait()

                        @pl.when(k + 1 < sec_p)
                        def _():
                            pltpu.async_copy(
                                h_refs[c].at[sidx.at[k + 1]], hbuf.at[o],
                                gsem.at[o])
                            pltpu.async_copy(
                                ws_refs[c].at[gp + 1], wbuf.at[o],
                                wsem.at[o])

                        pltpu.make_async_copy(
                            h_refs[c].at[sidx.at[k]], hbuf.at[s],
                            gsem.at[s]).wait()
                        pltpu.make_async_copy(
                            ws_refs[c].at[gp], wbuf.at[s], wsem.at[s]).wait()

                        def rowm(r, _):
                            for cc in range(8):
                                hbuf[s, r, pl.ds(cc * 16, 16)] = (
                                    hbuf[s, r, pl.ds(cc * 16, 16)]
                                    * wbuf[s, r, pl.ds(cc * 16, 16)])
                            return 0
                        lax.fori_loop(0, _P, rowm, 0)
                        pltpu.async_copy(
                            hbuf.at[s], acc.at[didx.at[k]], asem.at[s],
                            add=True)
                        return 0

                    lax.fori_loop(0, sec_p, piece, 0)
                    last = (sec_p - 1) % 2
                    pltpu.make_async_copy(
                        hbuf.at[last], acc.at[didx.at[sec_p - 1]],
                        asem.at[last]).wait()
                plsc.subcore_barrier()
                pltpu.sync_copy(
                    acc.at[pl.ds(sid * rows_dump, rows_dump), :],
                    out_refs[c].at[pl.ds(sid * rows_dump, rows_dump), :])
                plsc.subcore_barrier()

    mesh = plsc.VectorSubcoreMesh(core_axis_name="c", subcore_axis_name="s")
    return pl.kernel(
        body,
        out_type=[jax.ShapeDtypeStruct((_NP, _LANE), jnp.float32)] * nchunk,
        mesh=mesh,
        scratch_types=[
            pltpu.VMEM((sec_p, _P), jnp.int32),
            pltpu.VMEM((sec_p, _P), jnp.int32),
            pltpu.VMEM((2, _P, _LANE), jnp.float32),
            pltpu.VMEM((2, _P, _LANE), jnp.float32),
            pltpu.VMEM((16, _LANE), jnp.float32),
            pltpu.VMEM_SHARED((_NP, _LANE), jnp.float32),
            pltpu.SemaphoreType.DMA((2,)),
            pltpu.SemaphoreType.DMA((2,)),
            pltpu.SemaphoreType.DMA((2,)),
        ],
    )(*h_chunks, *ws_chunks, src3s, dst3s)


def _sc_hist(dst3s):
    pieces = _E // _NSC // _P
    nsec = 5
    sec_p = pieces // nsec
    rows_dump = _NP // _NSC

    def body(d4_ref, cnt_ref, didx, obuf, zbuf, acc):
        core = lax.axis_index("c")
        sid = lax.axis_index("s")

        @pl.when(core == 0)
        def _():
            _zero_zbuf(zbuf)

            def orow(r, _):
                for cc in range(8):
                    obuf[r, pl.ds(cc * 16, 16)] = jnp.ones((16,), jnp.float32)
                return 0
            lax.fori_loop(0, _P, orow, 0)

            def zrow(k, _):
                pltpu.sync_copy(
                    zbuf, acc.at[pl.ds(sid * rows_dump + k * 16, 16), :])
                return 0
            lax.fori_loop(0, rows_dump // 16, zrow, 0)
            plsc.subcore_barrier()

            for sec in range(nsec):
                pltpu.sync_copy(d4_ref.at[sid, sec], didx)

                def piece(k, _):
                    pltpu.sync_copy(obuf, acc.at[didx.at[k]], add=True)
                    return 0
                lax.fori_loop(0, sec_p, piece, 0)
            plsc.subcore_barrier()
            pltpu.sync_copy(
                acc.at[pl.ds(sid * rows_dump, rows_dump), :],
                cnt_ref.at[pl.ds(sid * rows_dump, rows_dump), :])

    mesh = plsc.VectorSubcoreMesh(core_axis_name="c", subcore_axis_name="s")
    return pl.kernel(
        body,
        out_type=jax.ShapeDtypeStruct((_NP, _LANE), jnp.float32),
        mesh=mesh,
        scratch_types=[
            pltpu.VMEM((sec_p, _P), jnp.int32),
            pltpu.VMEM((_P, _LANE), jnp.float32),
            pltpu.VMEM((16, _LANE), jnp.float32),
            pltpu.VMEM_SHARED((_NP, _LANE), jnp.float32),
        ],
    )(dst3s)


def _pad_to(x, n, axis):
    d = n - x.shape[axis]
    if d <= 0:
        return x
    cfg = [(0, 0)] * x.ndim
    cfg[axis] = (0, d)
    return jnp.pad(x, cfg)


def _mlp2_body(a_ref, w1_ref, b1_ref, w2_ref, b2_ref, o_ref):
    t = jnp.dot(a_ref[...], w1_ref[...], preferred_element_type=jnp.float32)
    t = jax.nn.relu(t + b1_ref[...])
    o = jnp.dot(t, w2_ref[...], preferred_element_type=jnp.float32)
    o_ref[...] = o + b2_ref[...]


def _mlp2(x, p, block_rows):
    w1, b1, w2, b2 = p
    rows = x.shape[0]
    grid = rows // block_rows
    kin = x.shape[1]
    return pl.pallas_call(
        _mlp2_body,
        grid=(grid,),
        in_specs=[
            pl.BlockSpec((block_rows, kin), lambda i: (i, 0)),
            pl.BlockSpec((w1.shape[0], w1.shape[1]), lambda i: (0, 0)),
            pl.BlockSpec((1, b1.shape[0]), lambda i: (0, 0)),
            pl.BlockSpec((w2.shape[0], w2.shape[1]), lambda i: (0, 0)),
            pl.BlockSpec((1, b2.shape[0]), lambda i: (0, 0)),
        ],
        out_specs=pl.BlockSpec((block_rows, w2.shape[1]), lambda i: (i, 0)),
        out_shape=jax.ShapeDtypeStruct((rows, w2.shape[1]), jnp.float32),
    )(x, w1, b1[None, :], w2, b2[None, :])


def _edge_ws_body(ea_ref, xs_ref, xd_ref, sh_ref, w1_ref, b1_ref, w2_ref,
                  b2_ref, wsh_ref, *o_refs):
    rows = ea_ref.shape[0]
    bp = rows // _P
    w1 = w1_ref[...]
    t = jnp.dot(ea_ref[...], w1[0:_NS], preferred_element_type=jnp.float32)
    xs = xs_ref[...].reshape(rows, _LANE)[:, :_NS]
    xd = xd_ref[...].reshape(rows, _LANE)[:, :_NS]
    t += jnp.dot(xs, w1[_NS:2 * _NS], preferred_element_type=jnp.float32)
    t += jnp.dot(xd, w1[2 * _NS:3 * _NS], preferred_element_type=jnp.float32)
    t = jax.nn.relu(t + b1_ref[...])
    w = jnp.dot(t, w2_ref[...], preferred_element_type=jnp.float32) + b2_ref[...]
    sh = jnp.dot(sh_ref[...], wsh_ref[...], preferred_element_type=jnp.float32)
    ws = w * sh
    for c, o_ref in enumerate(o_refs):
        o_ref[...] = ws[:, c * _LANE:(c + 1) * _LANE].reshape(bp, _P, _LANE)


def _edge_ws(ea, xs3, xd3, shp, w1, b1, w2, b2, wsh, block_rows):
    nchunk = w2.shape[1] // _LANE
    grid = _E // block_rows
    bp = block_rows // _P
    outs = [jax.ShapeDtypeStruct((_E // _P, _P, _LANE), jnp.float32)
            for _ in range(nchunk)]
    return pl.pallas_call(
        _edge_ws_body,
        grid=(grid,),
        in_specs=[
            pl.BlockSpec((block_rows, _NS), lambda i: (i, 0)),
            pl.BlockSpec((bp, _P, _LANE), lambda i: (i, 0, 0)),
            pl.BlockSpec((bp, _P, _LANE), lambda i: (i, 0, 0)),
            pl.BlockSpec((block_rows, 16), lambda i: (i, 0)),
            pl.BlockSpec((3 * _NS, 3 * _NS), lambda i: (0, 0)),
            pl.BlockSpec((1, 3 * _NS), lambda i: (0, 0)),
            pl.BlockSpec((3 * _NS, w2.shape[1]), lambda i: (0, 0)),
            pl.BlockSpec((1, w2.shape[1]), lambda i: (0, 0)),
            pl.BlockSpec((16, w2.shape[1]), lambda i: (0, 0)),
        ],
        out_specs=[pl.BlockSpec((bp, _P, _LANE), lambda i: (i, 0, 0))
                   for _ in range(nchunk)],
        out_shape=outs,
    )(ea, xs3, xd3, shp, w1, b1[None, :], w2, b2[None, :], wsh)


def _h_body(x_ref, wlin_ref, *o_refs):
    h = jnp.dot(x_ref[...], wlin_ref[...], preferred_element_type=jnp.float32)
    for c, o_ref in enumerate(o_refs):
        o_ref[...] = h[:, c * _LANE:(c + 1) * _LANE]


def _h_chunks(x, wlin, block_rows):
    nchunk = wlin.shape[1] // _LANE
    grid = _N // block_rows
    outs = [jax.ShapeDtypeStruct((_N, _LANE), jnp.float32) for _ in range(nchunk)]
    return pl.pallas_call(
        _h_body,
        grid=(grid,),
        in_specs=[
            pl.BlockSpec((block_rows, x.shape[1]), lambda i: (i, 0)),
            pl.BlockSpec((wlin.shape[0], wlin.shape[1]), lambda i: (0, 0)),
        ],
        out_specs=[pl.BlockSpec((block_rows, _LANE), lambda i: (i, 0))
                   for _ in range(nchunk)],
        out_shape=outs,
    )(x, wlin)


def _residual(x, dpad, r):
    xw = x.shape[1]
    if xw >= dpad:
        res = x[:, :dpad]
    else:
        res = jnp.pad(x, ((0, 0), (0, dpad - xw)))
    col = jax.lax.broadcasted_iota(jnp.int32, res.shape, 1)
    return jnp.where(col < r, res, 0.0)


def _stats_body(cnt_ref, x_ref, *refs, r):
    i = pl.program_id(0)
    nchunk = len(refs) - 2
    os_refs, (sum_ref, sq_ref) = refs[:nchunk], refs[nchunk:]
    dpad = nchunk * _LANE
    osum = jnp.concatenate([rf[...] for rf in os_refs], axis=1)
    otp = osum / jnp.maximum(cnt_ref[:, 0:1], 1.0)
    out = otp + _residual(x_ref[...], dpad, r)

    @pl.when(i == 0)
    def _():
        sum_ref[...] = jnp.zeros_like(sum_ref)
        sq_ref[...] = jnp.zeros_like(sq_ref)

    sum_ref[...] += jnp.broadcast_to(jnp.sum(out, 0, keepdims=True), sum_ref.shape)
    sq_ref[...] += jnp.broadcast_to(jnp.sum(out * out, 0, keepdims=True), sq_ref.shape)


def _norm_body(cnt_ref, x_ref, *refs, r):
    nchunk = len(refs) - 4
    os_refs = refs[:nchunk]
    sum_ref, sq_ref, y_ref, y32_ref = refs[nchunk:]
    dpad = nchunk * _LANE
    osum = jnp.concatenate([rf[...] for rf in os_refs], axis=1)
    otp = osum / jnp.maximum(cnt_ref[:, 0:1], 1.0)
    out = otp + _residual(x_ref[...], dpad, r)
    mean = sum_ref[0:1, :] / _N
    var = sq_ref[0:1, :] / _N - mean * mean
    y = (out - mean) * jax.lax.rsqrt(var + 1e-5)
    y_ref[...] = y
    y32_ref[...] = y[:, :_LANE]


def _finalize(osum_chunks, cnt, x, r, block_rows):
    nchunk = len(osum_chunks)
    dpad = nchunk * _LANE
    grid = _N // block_rows
    xw = x.shape[1]
    common_in = [
        pl.BlockSpec((block_rows, _LANE), lambda i: (i, 0)),
        pl.BlockSpec((block_rows, xw), lambda i: (i, 0)),
    ] + [pl.BlockSpec((block_rows, _LANE), lambda i: (i, 0)) for _ in range(nchunk)]
    sums = pl.pallas_call(
        functools.partial(_stats_body, r=r),
        grid=(grid,),
        in_specs=common_in,
        out_specs=[pl.BlockSpec((8, dpad), lambda i: (0, 0))] * 2,
        out_shape=[jax.ShapeDtypeStruct((8, dpad), jnp.float32)] * 2,
    )(cnt, x, *osum_chunks)
    y, y32 = pl.pallas_call(
        functools.partial(_norm_body, r=r),
        grid=(grid,),
        in_specs=common_in + [pl.BlockSpec((8, dpad), lambda i: (0, 0))] * 2,
        out_specs=[
            pl.BlockSpec((block_rows, dpad), lambda i: (i, 0)),
            pl.BlockSpec((block_rows, _LANE), lambda i: (i, 0)),
        ],
        out_shape=[
            jax.ShapeDtypeStruct((_N, dpad), jnp.float32),
            jax.ShapeDtypeStruct((_N, _LANE), jnp.float32),
        ],
    )(cnt, x, *osum_chunks, *sums)
    return y, y32


_DOUT_TRUE = [288, 544, 576, 483]
_DOUT_PAD = [384, 640, 640, 512]
_DIN_TRUE = [32, 288, 544, 576]


def kernel(node_attr, edge_index, edge_attr, edge_sh, node_mlp, edge_mlp, layers):
    src = edge_index[0]
    dst = edge_index[1]
    nt = _NC * _NSC
    src3g = src.reshape(nt, _E // nt // _P, _P)
    dst3g = dst.reshape(nt, _E // nt // _P, _P)
    pieces_t = _E // _NSC // _P
    src3s = src.reshape(_NSC, 5, pieces_t // 5, _P)
    dst3s = dst.reshape(_NSC, 5, pieces_t // 5, _P)
    shp = _pad_to(edge_sh, 16, 1)

    nw1, nb1, nw2, nb2 = node_mlp
    node_mlp_p = (nw1, nb1, _pad_to(nw2, _LANE, 1), _pad_to(nb2, _LANE, 0))
    x = _mlp2(node_attr, node_mlp_p, 1000)
    ea = _mlp2(edge_attr, edge_mlp, 2000)
    x32 = x

    cnt = _sc_hist(dst3s)

    for li, layer in enumerate(layers):
        dpad = _DOUT_PAD[li]
        dtrue = _DOUT_TRUE[li]
        w2 = _pad_to(layer['fcW2'], dpad, 1)
        b2 = _pad_to(layer['fcb2'], dpad, 0)
        wsh = _pad_to(layer['Wsh'], 16, 0)
        wsh = _pad_to(wsh, dpad, 1)
        wlin = _pad_to(layer['Wlin'], dpad, 1)
        wlin = _pad_to(wlin, x.shape[1], 0)

        xs3, xd3 = _sc_gather32(x32, src3g, dst3g)
        ws_chunks = _edge_ws(ea, xs3, xd3, shp, layer['fcW1'], layer['fcb1'],
                             w2, b2, wsh, 1600)
        h_chunks = _h_chunks(x, wlin, 2000)

        osum_chunks = _sc_scatter(h_chunks, ws_chunks, src3s, dst3s)

        r = min(_DIN_TRUE[li], dtrue)
        x, x32 = _finalize(osum_chunks, cnt, x, r, 1000)

    return (x[:, :483], edge_index)


# speedup vs baseline: 1.3751x; 1.3751x over previous
import functools
import jax
import jax.numpy as jnp
from jax import lax
from jax.experimental import pallas as pl
from jax.experimental.pallas import tpu as pltpu
from jax.experimental.pallas import tpu_sc as plsc

_N = 10000
_E = 160000
_NS = 32
_LANE = 128
_NP = 10240
_P = 100
_NC = 2
_NSC = 16


def _zero_zbuf(zbuf):
    for rr in range(zbuf.shape[0]):
        for cc in range(zbuf.shape[1] // 16):
            zbuf[rr, pl.ds(cc * 16, 16)] = jnp.zeros((16,), jnp.float32)


def _sc_gather32(x32, src3g, dst3g):
    rows_per_tile = _E // (_NC * _NSC) // _P

    def body(x32_ref, s3_ref, d3_ref, xs_ref, xd_ref, sidx, didx, sbuf, dbuf,
             gs, gd, ps, pd):
        core = lax.axis_index("c")
        sid = lax.axis_index("s")
        wid = sid * _NC + core
        row0 = wid * rows_per_tile
        pltpu.sync_copy(s3_ref.at[wid], sidx)
        pltpu.sync_copy(d3_ref.at[wid], didx)
        pltpu.async_copy(x32_ref.at[sidx.at[0]], sbuf.at[0], gs.at[0])
        pltpu.async_copy(x32_ref.at[didx.at[0]], dbuf.at[0], gd.at[0])

        def piece(k, _):
            s = lax.rem(k, 2)
            o = 1 - s

            @pl.when(k >= 1)
            def _():
                pltpu.make_async_copy(
                    sbuf.at[o], xs_ref.at[row0 + k - 1], ps.at[o]).wait()
                pltpu.make_async_copy(
                    dbuf.at[o], xd_ref.at[row0 + k - 1], pd.at[o]).wait()

            @pl.when(k + 1 < rows_per_tile)
            def _():
                pltpu.async_copy(
                    x32_ref.at[sidx.at[k + 1]], sbuf.at[o], gs.at[o])
                pltpu.async_copy(
                    x32_ref.at[didx.at[k + 1]], dbuf.at[o], gd.at[o])

            pltpu.make_async_copy(
                x32_ref.at[sidx.at[k]], sbuf.at[s], gs.at[s]).wait()
            pltpu.make_async_copy(
                x32_ref.at[didx.at[k]], dbuf.at[s], gd.at[s]).wait()
            pltpu.async_copy(sbuf.at[s], xs_ref.at[row0 + k], ps.at[s])
            pltpu.async_copy(dbuf.at[s], xd_ref.at[row0 + k], pd.at[s])
            return 0

        lax.fori_loop(0, rows_per_tile, piece, 0)
        last = (rows_per_tile - 1) % 2
        pltpu.make_async_copy(
            sbuf.at[last], xs_ref.at[row0 + rows_per_tile - 1],
            ps.at[last]).wait()
        pltpu.make_async_copy(
            dbuf.at[last], xd_ref.at[row0 + rows_per_tile - 1],
            pd.at[last]).wait()

    mesh = plsc.VectorSubcoreMesh(core_axis_name="c", subcore_axis_name="s")
    return pl.kernel(
        body,
        out_type=[jax.ShapeDtypeStruct((_E // _P, _P, _LANE), jnp.float32)] * 2,
        mesh=mesh,
        scratch_types=[
            pltpu.VMEM((rows_per_tile, _P), jnp.int32),
            pltpu.VMEM((rows_per_tile, _P), jnp.int32),
            pltpu.VMEM((2, _P, _LANE), jnp.float32),
            pltpu.VMEM((2, _P, _LANE), jnp.float32),
            pltpu.SemaphoreType.DMA((2,)),
            pltpu.SemaphoreType.DMA((2,)),
            pltpu.SemaphoreType.DMA((2,)),
            pltpu.SemaphoreType.DMA((2,)),
        ],
    )(x32, src3g, dst3g)


def _sc_scatter(h_chunks, ws_chunks, src3s, dst3s):
    nchunk = len(h_chunks)
    pieces = _E // _NSC // _P
    half_p = pieces // 2
    rows_dump = _NP // _NSC

    def body(*refs):
        h_refs = refs[:nchunk]
        ws_refs = refs[nchunk:2 * nchunk]
        s4_ref, d4_ref = refs[2 * nchunk:2 * nchunk + 2]
        out_refs = refs[2 * nchunk + 2:3 * nchunk + 2]
        sidx, didx, hbuf, wbuf, zbuf, acc, sem = refs[3 * nchunk + 2:]
        core = lax.axis_index("c")
        sid = lax.axis_index("s")
        _zero_zbuf(zbuf)

        for c in range(nchunk):
            @pl.when(core == (c % _NC))
            def _(c=c):
                def zrow(k, _):
                    pltpu.sync_copy(
                        zbuf, acc.at[pl.ds(sid * rows_dump + k * 16, 16), :])
                    return 0
                lax.fori_loop(0, rows_dump // 16, zrow, 0)
                plsc.subcore_barrier()

                for half in range(2):
                    pltpu.sync_copy(s4_ref.at[sid, half], sidx)
                    pltpu.sync_copy(d4_ref.at[sid, half], didx)

                    def piece(k, _, half=half):
                        gp = sid * pieces + half * half_p + k
                        pltpu.async_copy(
                            h_refs[c].at[sidx.at[k]], hbuf, sem).wait()
                        pltpu.sync_copy(ws_refs[c].at[gp], wbuf)

                        def rowm(r, _):
                            for cc in range(8):
                                hbuf[r, pl.ds(cc * 16, 16)] = (
                                    hbuf[r, pl.ds(cc * 16, 16)]
                                    * wbuf[r, pl.ds(cc * 16, 16)])
                            return 0
                        lax.fori_loop(0, _P, rowm, 0)
                        pltpu.sync_copy(hbuf, acc.at[didx.at[k]], add=True)
                        return 0

                    lax.fori_loop(0, half_p, piece, 0)
                plsc.subcore_barrier()
                pltpu.sync_copy(
                    acc.at[pl.ds(sid * rows_dump, rows_dump), :],
                    out_refs[c].at[pl.ds(sid * rows_dump, rows_dump), :])
                plsc.subcore_barrier()

    mesh = plsc.VectorSubcoreMesh(core_axis_name="c", subcore_axis_name="s")
    return pl.kernel(
        body,
        out_type=[jax.ShapeDtypeStruct((_NP, _LANE), jnp.float32)] * nchunk,
        mesh=mesh,
        scratch_types=[
            pltpu.VMEM((half_p, _P), jnp.int32),
            pltpu.VMEM((half_p, _P), jnp.int32),
            pltpu.VMEM((_P, _LANE), jnp.float32),
            pltpu.VMEM((_P, _LANE), jnp.float32),
            pltpu.VMEM((16, _LANE), jnp.float32),
            pltpu.VMEM_SHARED((_NP, _LANE), jnp.float32),
            pltpu.SemaphoreType.DMA,
        ],
    )(*h_chunks, *ws_chunks, src3s, dst3s)


def _sc_hist(dst3s):
    pieces = _E // _NSC // _P
    half_p = pieces // 2
    rows_dump = _NP // _NSC

    def body(d4_ref, cnt_ref, didx, obuf, zbuf, acc):
        core = lax.axis_index("c")
        sid = lax.axis_index("s")

        @pl.when(core == 0)
        def _():
            _zero_zbuf(zbuf)

            def orow(r, _):
                for cc in range(8):
                    obuf[r, pl.ds(cc * 16, 16)] = jnp.ones((16,), jnp.float32)
                return 0
            lax.fori_loop(0, _P, orow, 0)

            def zrow(k, _):
                pltpu.sync_copy(
                    zbuf, acc.at[pl.ds(sid * rows_dump + k * 16, 16), :])
                return 0
            lax.fori_loop(0, rows_dump // 16, zrow, 0)
            plsc.subcore_barrier()

            for half in range(2):
                pltpu.sync_copy(d4_ref.at[sid, half], didx)

                def piece(k, _):
                    pltpu.sync_copy(obuf, acc.at[didx.at[k]], add=True)
                    return 0
                lax.fori_loop(0, half_p, piece, 0)
            plsc.subcore_barrier()
            pltpu.sync_copy(
                acc.at[pl.ds(sid * rows_dump, rows_dump), :],
                cnt_ref.at[pl.ds(sid * rows_dump, rows_dump), :])

    mesh = plsc.VectorSubcoreMesh(core_axis_name="c", subcore_axis_name="s")
    return pl.kernel(
        body,
        out_type=jax.ShapeDtypeStruct((_NP, _LANE), jnp.float32),
        mesh=mesh,
        scratch_types=[
            pltpu.VMEM((half_p, _P), jnp.int32),
            pltpu.VMEM((_P, _LANE), jnp.float32),
            pltpu.VMEM((16, _LANE), jnp.float32),
            pltpu.VMEM_SHARED((_NP, _LANE), jnp.float32),
        ],
    )(dst3s)


def _pad_to(x, n, axis):
    d = n - x.shape[axis]
    if d <= 0:
        return x
    cfg = [(0, 0)] * x.ndim
    cfg[axis] = (0, d)
    return jnp.pad(x, cfg)


def _mlp2_body(a_ref, w1_ref, b1_ref, w2_ref, b2_ref, o_ref):
    t = jnp.dot(a_ref[...], w1_ref[...], preferred_element_type=jnp.float32)
    t = jax.nn.relu(t + b1_ref[...])
    o = jnp.dot(t, w2_ref[...], preferred_element_type=jnp.float32)
    o_ref[...] = o + b2_ref[...]


def _mlp2(x, p, block_rows):
    w1, b1, w2, b2 = p
    rows = x.shape[0]
    grid = rows // block_rows
    kin = x.shape[1]
    return pl.pallas_call(
        _mlp2_body,
        grid=(grid,),
        in_specs=[
            pl.BlockSpec((block_rows, kin), lambda i: (i, 0)),
            pl.BlockSpec((w1.shape[0], w1.shape[1]), lambda i: (0, 0)),
            pl.BlockSpec((1, b1.shape[0]), lambda i: (0, 0)),
            pl.BlockSpec((w2.shape[0], w2.shape[1]), lambda i: (0, 0)),
            pl.BlockSpec((1, b2.shape[0]), lambda i: (0, 0)),
        ],
        out_specs=pl.BlockSpec((block_rows, w2.shape[1]), lambda i: (i, 0)),
        out_shape=jax.ShapeDtypeStruct((rows, w2.shape[1]), jnp.float32),
    )(x, w1, b1[None, :], w2, b2[None, :])


def _edge_ws_body(ea_ref, xs_ref, xd_ref, sh_ref, w1_ref, b1_ref, w2_ref,
                  b2_ref, wsh_ref, *o_refs):
    rows = ea_ref.shape[0]
    bp = rows // _P
    w1 = w1_ref[...]
    t = jnp.dot(ea_ref[...], w1[0:_NS], preferred_element_type=jnp.float32)
    xs = xs_ref[...].reshape(rows, _LANE)[:, :_NS]
    xd = xd_ref[...].reshape(rows, _LANE)[:, :_NS]
    t += jnp.dot(xs, w1[_NS:2 * _NS], preferred_element_type=jnp.float32)
    t += jnp.dot(xd, w1[2 * _NS:3 * _NS], preferred_element_type=jnp.float32)
    t = jax.nn.relu(t + b1_ref[...])
    w = jnp.dot(t, w2_ref[...], preferred_element_type=jnp.float32) + b2_ref[...]
    sh = jnp.dot(sh_ref[...], wsh_ref[...], preferred_element_type=jnp.float32)
    ws = w * sh
    for c, o_ref in enumerate(o_refs):
        o_ref[...] = ws[:, c * _LANE:(c + 1) * _LANE].reshape(bp, _P, _LANE)


def _edge_ws(ea, xs3, xd3, shp, w1, b1, w2, b2, wsh, block_rows):
    nchunk = w2.shape[1] // _LANE
    grid = _E // block_rows
    bp = block_rows // _P
    outs = [jax.ShapeDtypeStruct((_E // _P, _P, _LANE), jnp.float32)
            for _ in range(nchunk)]
    return pl.pallas_call(
        _edge_ws_body,
        grid=(grid,),
        in_specs=[
            pl.BlockSpec((block_rows, _NS), lambda i: (i, 0)),
            pl.BlockSpec((bp, _P, _LANE), lambda i: (i, 0, 0)),
            pl.BlockSpec((bp, _P, _LANE), lambda i: (i, 0, 0)),
            pl.BlockSpec((block_rows, 16), lambda i: (i, 0)),
            pl.BlockSpec((3 * _NS, 3 * _NS), lambda i: (0, 0)),
            pl.BlockSpec((1, 3 * _NS), lambda i: (0, 0)),
            pl.BlockSpec((3 * _NS, w2.shape[1]), lambda i: (0, 0)),
            pl.BlockSpec((1, w2.shape[1]), lambda i: (0, 0)),
            pl.BlockSpec((16, w2.shape[1]), lambda i: (0, 0)),
        ],
        out_specs=[pl.BlockSpec((bp, _P, _LANE), lambda i: (i, 0, 0))
                   for _ in range(nchunk)],
        out_shape=outs,
    )(ea, xs3, xd3, shp, w1, b1[None, :], w2, b2[None, :], wsh)


def _h_body(x_ref, wlin_ref, *o_refs):
    h = jnp.dot(x_ref[...], wlin_ref[...], preferred_element_type=jnp.float32)
    for c, o_ref in enumerate(o_refs):
        o_ref[...] = h[:, c * _LANE:(c + 1) * _LANE]


def _h_chunks(x, wlin, block_rows):
    nchunk = wlin.shape[1] // _LANE
    grid = _N // block_rows
    outs = [jax.ShapeDtypeStruct((_N, _LANE), jnp.float32) for _ in range(nchunk)]
    return pl.pallas_call(
        _h_body,
        grid=(grid,),
        in_specs=[
            pl.BlockSpec((block_rows, x.shape[1]), lambda i: (i, 0)),
            pl.BlockSpec((wlin.shape[0], wlin.shape[1]), lambda i: (0, 0)),
        ],
        out_specs=[pl.BlockSpec((block_rows, _LANE), lambda i: (i, 0))
                   for _ in range(nchunk)],
        out_shape=outs,
    )(x, wlin)


def _residual(x, dpad, r):
    xw = x.shape[1]
    if xw >= dpad:
        res = x[:, :dpad]
    else:
        res = jnp.pad(x, ((0, 0), (0, dpad - xw)))
    col = jax.lax.broadcasted_iota(jnp.int32, res.shape, 1)
    return jnp.where(col < r, res, 0.0)


def _stats_body(cnt_ref, x_ref, *refs, r):
    i = pl.program_id(0)
    nchunk = len(refs) - 2
    os_refs, (sum_ref, sq_ref) = refs[:nchunk], refs[nchunk:]
    dpad = nchunk * _LANE
    osum = jnp.concatenate([rf[...] for rf in os_refs], axis=1)
    otp = osum / jnp.maximum(cnt_ref[:, 0:1], 1.0)
    out = otp + _residual(x_ref[...], dpad, r)

    @pl.when(i == 0)
    def _():
        sum_ref[...] = jnp.zeros_like(sum_ref)
        sq_ref[...] = jnp.zeros_like(sq_ref)

    sum_ref[...] += jnp.broadcast_to(jnp.sum(out, 0, keepdims=True), sum_ref.shape)
    sq_ref[...] += jnp.broadcast_to(jnp.sum(out * out, 0, keepdims=True), sq_ref.shape)


def _norm_body(cnt_ref, x_ref, *refs, r):
    nchunk = len(refs) - 4
    os_refs = refs[:nchunk]
    sum_ref, sq_ref, y_ref, y32_ref = refs[nchunk:]
    dpad = nchunk * _LANE
    osum = jnp.concatenate([rf[...] for rf in os_refs], axis=1)
    otp = osum / jnp.maximum(cnt_ref[:, 0:1], 1.0)
    out = otp + _residual(x_ref[...], dpad, r)
    mean = sum_ref[0:1, :] / _N
    var = sq_ref[0:1, :] / _N - mean * mean
    y = (out - mean) * jax.lax.rsqrt(var + 1e-5)
    y_ref[...] = y
    y32_ref[...] = y[:, :_LANE]


def _finalize(osum_chunks, cnt, x, r, block_rows):
    nchunk = len(osum_chunks)
    dpad = nchunk * _LANE
    grid = _N // block_rows
    xw = x.shape[1]
    common_in = [
        pl.BlockSpec((block_rows, _LANE), lambda i: (i, 0)),
        pl.BlockSpec((block_rows, xw), lambda i: (i, 0)),
    ] + [pl.BlockSpec((block_rows, _LANE), lambda i: (i, 0)) for _ in range(nchunk)]
    sums = pl.pallas_call(
        functools.partial(_stats_body, r=r),
        grid=(grid,),
        in_specs=common_in,
        out_specs=[pl.BlockSpec((8, dpad), lambda i: (0, 0))] * 2,
        out_shape=[jax.ShapeDtypeStruct((8, dpad), jnp.float32)] * 2,
    )(cnt, x, *osum_chunks)
    y, y32 = pl.pallas_call(
        functools.partial(_norm_body, r=r),
        grid=(grid,),
        in_specs=common_in + [pl.BlockSpec((8, dpad), lambda i: (0, 0))] * 2,
        out_specs=[
            pl.BlockSpec((block_rows, dpad), lambda i: (i, 0)),
            pl.BlockSpec((block_rows, _LANE), lambda i: (i, 0)),
        ],
        out_shape=[
            jax.ShapeDtypeStruct((_N, dpad), jnp.float32),
            jax.ShapeDtypeStruct((_N, _LANE), jnp.float32),
        ],
    )(cnt, x, *osum_chunks, *sums)
    return y, y32


_DOUT_TRUE = [288, 544, 576, 483]
_DOUT_PAD = [384, 640, 640, 512]
_DIN_TRUE = [32, 288, 544, 576]


def kernel(node_attr, edge_index, edge_attr, edge_sh, node_mlp, edge_mlp, layers):
    src = edge_index[0]
    dst = edge_index[1]
    nt = _NC * _NSC
    src3g = src.reshape(nt, _E // nt // _P, _P)
    dst3g = dst.reshape(nt, _E // nt // _P, _P)
    pieces_t = _E // _NSC // _P
    src3s = src.reshape(_NSC, 2, pieces_t // 2, _P)
    dst3s = dst.reshape(_NSC, 2, pieces_t // 2, _P)
    shp = _pad_to(edge_sh, 16, 1)

    nw1, nb1, nw2, nb2 = node_mlp
    node_mlp_p = (nw1, nb1, _pad_to(nw2, _LANE, 1), _pad_to(nb2, _LANE, 0))
    x = _mlp2(node_attr, node_mlp_p, 1000)
    ea = _mlp2(edge_attr, edge_mlp, 2000)
    x32 = x

    cnt = _sc_hist(dst3s)

    for li, layer in enumerate(layers):
        dpad = _DOUT_PAD[li]
        dtrue = _DOUT_TRUE[li]
        w2 = _pad_to(layer['fcW2'], dpad, 1)
        b2 = _pad_to(layer['fcb2'], dpad, 0)
        wsh = _pad_to(layer['Wsh'], 16, 0)
        wsh = _pad_to(wsh, dpad, 1)
        wlin = _pad_to(layer['Wlin'], dpad, 1)
        wlin = _pad_to(wlin, x.shape[1], 0)

        xs3, xd3 = _sc_gather32(x32, src3g, dst3g)
        ws_chunks = _edge_ws(ea, xs3, xd3, shp, layer['fcW1'], layer['fcb1'],
                             w2, b2, wsh, 1600)
        h_chunks = _h_chunks(x, wlin, 2000)

        osum_chunks = _sc_scatter(h_chunks, ws_chunks, src3s, dst3s)

        r = min(_DIN_TRUE[li], dtrue)
        x, x32 = _finalize(osum_chunks, cnt, x, r, 1000)

    return (x[:, :483], edge_index)
